# baseline (speedup 1.0000x reference)
import functools

import jax
import jax.numpy as jnp
from jax import lax
from jax.experimental import pallas as pl
from jax.experimental.pallas import tpu as pltpu
from jax.experimental.pallas import tpu_sc as plsc

N_NODES = 10000
D = 128
NC = 2
NS = 16
NW = NC * NS
CHUNK = 128
NBUF = 4
N_ACC = 10240
PAD_DST = N_NODES + 8
SLC = N_ACC // NS

_mesh = plsc.VectorSubcoreMesh(core_axis_name="c", subcore_axis_name="s")


def _make_deg_kernel(nch):
    LAG = 8

    @functools.partial(
        pl.kernel,
        out_type=[jax.ShapeDtypeStruct((N_ACC,), jnp.float32),
                  jax.ShapeDtypeStruct((N_ACC,), jnp.float32)],
        mesh=_mesh,
        scratch_types=[
            pltpu.VMEM((nch, CHUNK), jnp.int32),
            pltpu.VMEM((CHUNK,), jnp.float32),
            pltpu.VMEM((SLC,), jnp.float32),
            pltpu.VMEM_SHARED((N_ACC,), jnp.float32),
            pltpu.SemaphoreType.DMA,
        ],
    )
    def deg_kernel(dst3_hbm, out0_hbm, out1_hbm, didx2, ones_v, zbuf, sdeg,
                   sem):
        c = lax.axis_index("c")
        s = lax.axis_index("s")
        wid = c * NS + s

        def _zero(i, _):
            zbuf[pl.ds(i * 16, 16)] = jnp.zeros((16,), jnp.float32)
            return 0
        lax.fori_loop(0, SLC // 16, _zero, 0)
        for j in range(CHUNK // 16):
            ones_v[pl.ds(j * 16, 16)] = jnp.ones((16,), jnp.float32)
        pltpu.sync_copy(dst3_hbm.at[pl.ds(wid * nch, nch)], didx2)
        pltpu.sync_copy(zbuf, sdeg.at[pl.ds(s * SLC, SLC)])
        plsc.subcore_barrier()

        def _fire(i, _):
            pltpu.async_copy(ones_v, sdeg.at[didx2.at[i]], sem, add=True)

            @pl.when(i >= LAG)
            def _():
                pltpu.make_async_copy(ones_v, sdeg.at[didx2.at[0]],
                                      sem).wait()
            return 0
        lax.fori_loop(0, nch, _fire, 0)

        def _drain(i, _):
            pltpu.make_async_copy(ones_v, sdeg.at[didx2.at[0]], sem).wait()
            return 0
        lax.fori_loop(0, min(LAG, nch), _drain, 0)
        plsc.subcore_barrier()

        pltpu.sync_copy(sdeg.at[pl.ds(s * SLC, SLC)], zbuf)

        @pl.when(c == 0)
        def _():
            pltpu.sync_copy(zbuf, out0_hbm.at[pl.ds(s * SLC, SLC)])

        @pl.when(c == 1)
        def _():
            pltpu.sync_copy(zbuf, out1_hbm.at[pl.ds(s * SLC, SLC)])

    return deg_kernel


def _make_scatter_kernel(nch):
    assert nch % NBUF == 0
    epw = nch * CHUNK

    @functools.partial(
        pl.kernel,
        out_type=[jax.ShapeDtypeStruct((N_ACC, D), jnp.float32),
                  jax.ShapeDtypeStruct((N_ACC, D), jnp.float32)],
        mesh=_mesh,
        scratch_types=[
            pltpu.VMEM((CHUNK,), jnp.int32),
            pltpu.VMEM((CHUNK,), jnp.int32),
            pltpu.VMEM((CHUNK, D), jnp.float32),
            pltpu.VMEM((CHUNK, D), jnp.float32),
            pltpu.VMEM_SHARED((N_ACC, D), jnp.float32),
        ],
    )
    def scatter_kernel(g_hbm, src_hbm, dst_hbm, out0_hbm, out1_hbm,
                       sidx, didx, rows, zbuf, acc):
        c = lax.axis_index("c")
        s = lax.axis_index("s")
        wid = c * NS + s
        base = wid * epw

        def _zero(i, _):
            zbuf[i // 8, pl.ds((i % 8) * 16, 16)] = jnp.zeros((16,),
                                                              jnp.float32)
            return 0
        lax.fori_loop(0, CHUNK * (D // 16), _zero, 0)

        for j in range(SLC // CHUNK):
            pltpu.sync_copy(zbuf, acc.at[pl.ds(s * SLC + j * CHUNK, CHUNK)])
        plsc.subcore_barrier()

        def _body(i, _):
            off = base + i * CHUNK
            pltpu.sync_copy(src_hbm.at[pl.ds(off, CHUNK)], sidx)
            pltpu.sync_copy(dst_hbm.at[pl.ds(off, CHUNK)], didx)
            pltpu.sync_copy(g_hbm.at[sidx], rows)
            pltpu.sync_copy(rows, acc.at[didx], add=True)
            return 0
        lax.fori_loop(0, nch, _body, 0)
        plsc.subcore_barrier()

        for j in range(SLC // CHUNK):
            r0 = s * SLC + j * CHUNK
            pltpu.sync_copy(acc.at[pl.ds(r0, CHUNK)], zbuf)

            @pl.when(c == 0)
            def _():
                pltpu.sync_copy(zbuf, out0_hbm.at[pl.ds(r0, CHUNK)])

            @pl.when(c == 1)
            def _():
                pltpu.sync_copy(zbuf, out1_hbm.at[pl.ds(r0, CHUNK)])

    return scatter_kernel


BLK = 400


def _matmul_body(x_ref, w_ref, degp_ref, g_ref):
    deg = degp_ref[:, 0] + degp_ref[:, 1] + 1.0
    dis = lax.rsqrt(deg)
    h = jnp.dot(x_ref[...], w_ref[...], preferred_element_type=jnp.float32)
    g_ref[...] = h * dis[:, None]


def _matmul(x, w, degp_t):
    return pl.pallas_call(
        _matmul_body,
        grid=(N_NODES // BLK,),
        in_specs=[
            pl.BlockSpec((BLK, D), lambda i: (i, 0)),
            pl.BlockSpec((D, D), lambda i: (0, 0)),
            pl.BlockSpec((BLK, NC), lambda i: (i, 0)),
        ],
        out_specs=pl.BlockSpec((BLK, D), lambda i: (i, 0)),
        out_shape=jax.ShapeDtypeStruct((N_NODES, D), jnp.float32),
    )(x, w, degp_t)


def _final_body(p0_ref, p1_ref, g_ref, degp_ref, b_ref, o_ref):
    deg = degp_ref[:, 0] + degp_ref[:, 1] + 1.0
    dis = lax.rsqrt(deg)
    o_ref[...] = (dis[:, None] * (p0_ref[...] + p1_ref[...] + g_ref[...])
                  + b_ref[...])


def _final(p0, p1, g, degp_t, b2d):
    return pl.pallas_call(
        _final_body,
        grid=(N_NODES // BLK,),
        in_specs=[
            pl.BlockSpec((BLK, D), lambda i: (i, 0)),
            pl.BlockSpec((BLK, D), lambda i: (i, 0)),
            pl.BlockSpec((BLK, D), lambda i: (i, 0)),
            pl.BlockSpec((BLK, NC), lambda i: (i, 0)),
            pl.BlockSpec((1, D), lambda i: (0, 0)),
        ],
        out_specs=pl.BlockSpec((BLK, D), lambda i: (i, 0)),
        out_shape=jax.ShapeDtypeStruct((N_NODES, D), jnp.float32),
    )(p0, p1, g, degp_t, b2d)


def kernel(node_features, adjacency_matrix, W, b):
    src = adjacency_matrix[0].astype(jnp.int32)
    dst = adjacency_matrix[1].astype(jnp.int32)
    n_edges = src.shape[0]
    quantum = NW * CHUNK * NBUF
    n_pad = (-n_edges) % quantum
    if n_pad:
        src = jnp.concatenate([src, jnp.zeros((n_pad,), jnp.int32)])
        dst = jnp.concatenate([dst, jnp.full((n_pad,), PAD_DST, jnp.int32)])
    nch = (n_edges + n_pad) // (NW * CHUNK)
    dst3 = dst.reshape(NW * nch, CHUNK)

    d0, d1 = _make_deg_kernel(nch)(dst3)
    degp_t = jnp.stack([d0[:N_NODES], d1[:N_NODES]], axis=1)
    g = _matmul(node_features, W, degp_t)
    p0, p1 = _make_scatter_kernel(nch)(g, src, dst)
    return _final(p0, p1, g, degp_t, b.reshape(1, D))


# speedup vs baseline: 1.1799x; 1.1799x over previous
import functools

import jax
import jax.numpy as jnp
from jax import lax
from jax.experimental import pallas as pl
from jax.experimental.pallas import tpu as pltpu
from jax.experimental.pallas import tpu_sc as plsc

N_NODES = 10000
D = 128
NC = 2
NS = 16
NW = NC * NS
CHUNK = 128
NBUF = 4
N_ACC = 10240
PAD_DST = N_NODES + 8
SLC = N_ACC // NS

_mesh = plsc.VectorSubcoreMesh(core_axis_name="c", subcore_axis_name="s")


def _make_deg_kernel(nch):
    LAG = 8

    @functools.partial(
        pl.kernel,
        out_type=[jax.ShapeDtypeStruct((N_ACC,), jnp.float32),
                  jax.ShapeDtypeStruct((N_ACC,), jnp.float32)],
        mesh=_mesh,
        scratch_types=[
            pltpu.VMEM((nch, CHUNK), jnp.int32),
            pltpu.VMEM((CHUNK,), jnp.float32),
            pltpu.VMEM((SLC,), jnp.float32),
            pltpu.VMEM_SHARED((N_ACC,), jnp.float32),
            pltpu.SemaphoreType.DMA,
        ],
    )
    def deg_kernel(dst3_hbm, out0_hbm, out1_hbm, didx2, ones_v, zbuf, sdeg,
                   sem):
        c = lax.axis_index("c")
        s = lax.axis_index("s")
        wid = c * NS + s

        def _zero(i, _):
            zbuf[pl.ds(i * 16, 16)] = jnp.zeros((16,), jnp.float32)
            return 0
        lax.fori_loop(0, SLC // 16, _zero, 0)
        for j in range(CHUNK // 16):
            ones_v[pl.ds(j * 16, 16)] = jnp.ones((16,), jnp.float32)
        pltpu.sync_copy(dst3_hbm.at[pl.ds(wid * nch, nch)], didx2)
        pltpu.sync_copy(zbuf, sdeg.at[pl.ds(s * SLC, SLC)])
        plsc.subcore_barrier()

        def _fire(i, _):
            pltpu.async_copy(ones_v, sdeg.at[didx2.at[i]], sem, add=True)

            @pl.when(i >= LAG)
            def _():
                pltpu.make_async_copy(ones_v, sdeg.at[didx2.at[0]],
                                      sem).wait()
            return 0
        lax.fori_loop(0, nch, _fire, 0)

        def _drain(i, _):
            pltpu.make_async_copy(ones_v, sdeg.at[didx2.at[0]], sem).wait()
            return 0
        lax.fori_loop(0, min(LAG, nch), _drain, 0)
        plsc.subcore_barrier()

        pltpu.sync_copy(sdeg.at[pl.ds(s * SLC, SLC)], zbuf)

        @pl.when(c == 0)
        def _():
            pltpu.sync_copy(zbuf, out0_hbm.at[pl.ds(s * SLC, SLC)])

        @pl.when(c == 1)
        def _():
            pltpu.sync_copy(zbuf, out1_hbm.at[pl.ds(s * SLC, SLC)])

    return deg_kernel


def _make_scatter_kernel(nch):
    assert nch % NBUF == 0

    @functools.partial(
        pl.kernel,
        out_type=[jax.ShapeDtypeStruct((N_ACC, D), jnp.float32),
                  jax.ShapeDtypeStruct((N_ACC, D), jnp.float32)],
        mesh=_mesh,
        scratch_types=[
            pltpu.VMEM((nch, CHUNK), jnp.int32),
            pltpu.VMEM((CHUNK,), jnp.int32),
            pltpu.VMEM((CHUNK,), jnp.int32),
            pltpu.VMEM((CHUNK,), jnp.int32),
            pltpu.VMEM((CHUNK,), jnp.int32),
            pltpu.VMEM((CHUNK, D), jnp.float32),
            pltpu.VMEM((CHUNK, D), jnp.float32),
            pltpu.VMEM_SHARED((N_ACC, D), jnp.float32),
            pltpu.SemaphoreType.DMA,
            pltpu.SemaphoreType.DMA,
            pltpu.SemaphoreType.DMA,
            pltpu.SemaphoreType.DMA,
            pltpu.SemaphoreType.DMA,
            pltpu.SemaphoreType.DMA,
            pltpu.SemaphoreType.DMA,
            pltpu.SemaphoreType.DMA,
        ],
    )
    def scatter_kernel(g_hbm, src3_hbm, dst3_hbm, out0_hbm, out1_hbm,
                       didx2, sx0, sx1, sx2, sx3, r0, r1, acc,
                       is0, is1, is2, is3, gs0, gs1, ss0, ss1):
        sidx = [sx0, sx1, sx2, sx3]
        rows = [r0, r1]
        isem = [is0, is1, is2, is3]
        gsem = [gs0, gs1]
        ssem = [ss0, ss1]
        c = lax.axis_index("c")
        s = lax.axis_index("s")
        wid = c * NS + s
        base = wid * nch

        def _zero(i, _):
            r0[i // 8, pl.ds((i % 8) * 16, 16)] = jnp.zeros((16,),
                                                            jnp.float32)
            return 0
        lax.fori_loop(0, CHUNK * (D // 16), _zero, 0)

        pltpu.sync_copy(dst3_hbm.at[pl.ds(base, nch)], didx2)
        for j in range(SLC // CHUNK):
            pltpu.sync_copy(r0, acc.at[pl.ds(s * SLC + j * CHUNK, CHUNK)])
        plsc.subcore_barrier()

        for k in range(4):
            pltpu.async_copy(src3_hbm.at[base + k], sidx[k], isem[k])

        def _wait_i(k):
            pltpu.make_async_copy(src3_hbm.at[base], sidx[k], isem[k]).wait()

        def _wait_g(k):
            pltpu.make_async_copy(g_hbm.at[sidx[0]], rows[k], gsem[k]).wait()

        def _wait_s(k):
            pltpu.make_async_copy(rows[k], acc.at[didx2.at[0]],
                                  ssem[k]).wait()

        def _outer(io, _):
            for k in range(4):
                i = io * 4 + k
                k2 = k % 2

                @pl.when(i >= 2)
                def _():
                    _wait_s(k2)

                @pl.when((i >= 2) & (i + 2 < nch))
                def _():
                    pltpu.async_copy(src3_hbm.at[base + i + 2],
                                     sidx[(k + 2) % 4], isem[(k + 2) % 4])
                _wait_i(k)
                pltpu.async_copy(g_hbm.at[sidx[k]], rows[k2], gsem[k2])

                @pl.when(i >= 1)
                def _():
                    _wait_g(1 - k2)
                    pltpu.async_copy(rows[1 - k2], acc.at[didx2.at[i - 1]],
                                     ssem[1 - k2], add=True)
            return 0
        lax.fori_loop(0, nch // 4, _outer, 0)

        _wait_g((nch - 1) % 2)
        pltpu.async_copy(rows[(nch - 1) % 2], acc.at[didx2.at[nch - 1]],
                         ssem[(nch - 1) % 2], add=True)
        _wait_s(0)
        _wait_s(1)
        plsc.subcore_barrier()

        for j in range(SLC // CHUNK):
            off = s * SLC + j * CHUNK
            pltpu.sync_copy(acc.at[pl.ds(off, CHUNK)], r0)

            @pl.when(c == 0)
            def _():
                pltpu.sync_copy(r0, out0_hbm.at[pl.ds(off, CHUNK)])

            @pl.when(c == 1)
            def _():
                pltpu.sync_copy(r0, out1_hbm.at[pl.ds(off, CHUNK)])

    return scatter_kernel


BLK = 400


def _matmul_body(x_ref, w_ref, degp_ref, g_ref):
    deg = degp_ref[:, 0] + degp_ref[:, 1] + 1.0
    dis = lax.rsqrt(deg)
    h = jnp.dot(x_ref[...], w_ref[...], preferred_element_type=jnp.float32)
    g_ref[...] = h * dis[:, None]


def _matmul(x, w, degp_t):
    return pl.pallas_call(
        _matmul_body,
        grid=(N_NODES // BLK,),
        in_specs=[
            pl.BlockSpec((BLK, D), lambda i: (i, 0)),
            pl.BlockSpec((D, D), lambda i: (0, 0)),
            pl.BlockSpec((BLK, NC), lambda i: (i, 0)),
        ],
        out_specs=pl.BlockSpec((BLK, D), lambda i: (i, 0)),
        out_shape=jax.ShapeDtypeStruct((N_NODES, D), jnp.float32),
    )(x, w, degp_t)


def _final_body(p0_ref, p1_ref, g_ref, degp_ref, b_ref, o_ref):
    deg = degp_ref[:, 0] + degp_ref[:, 1] + 1.0
    dis = lax.rsqrt(deg)
    o_ref[...] = (dis[:, None] * (p0_ref[...] + p1_ref[...] + g_ref[...])
                  + b_ref[...])


def _final(p0, p1, g, degp_t, b2d):
    return pl.pallas_call(
        _final_body,
        grid=(N_NODES // BLK,),
        in_specs=[
            pl.BlockSpec((BLK, D), lambda i: (i, 0)),
            pl.BlockSpec((BLK, D), lambda i: (i, 0)),
            pl.BlockSpec((BLK, D), lambda i: (i, 0)),
            pl.BlockSpec((BLK, NC), lambda i: (i, 0)),
            pl.BlockSpec((1, D), lambda i: (0, 0)),
        ],
        out_specs=pl.BlockSpec((BLK, D), lambda i: (i, 0)),
        out_shape=jax.ShapeDtypeStruct((N_NODES, D), jnp.float32),
    )(p0, p1, g, degp_t, b2d)


def kernel(node_features, adjacency_matrix, W, b):
    src = adjacency_matrix[0].astype(jnp.int32)
    dst = adjacency_matrix[1].astype(jnp.int32)
    n_edges = src.shape[0]
    quantum = NW * CHUNK * NBUF
    n_pad = (-n_edges) % quantum
    if n_pad:
        src = jnp.concatenate([src, jnp.zeros((n_pad,), jnp.int32)])
        dst = jnp.concatenate([dst, jnp.full((n_pad,), PAD_DST, jnp.int32)])
    nch = (n_edges + n_pad) // (NW * CHUNK)
    src3 = src.reshape(NW * nch, CHUNK)
    dst3 = dst.reshape(NW * nch, CHUNK)

    d0, d1 = _make_deg_kernel(nch)(dst3)
    degp_t = jnp.stack([d0[:N_NODES], d1[:N_NODES]], axis=1)
    g = _matmul(node_features, W, degp_t)
    p0, p1 = _make_scatter_kernel(nch)(g, src3, dst3)
    return _final(p0, p1, g, degp_t, b.reshape(1, D))


# speedup vs baseline: 1.4570x; 1.2349x over previous
import functools

import jax
import jax.numpy as jnp
from jax import lax
from jax.experimental import pallas as pl
from jax.experimental.pallas import tpu as pltpu
from jax.experimental.pallas import tpu_sc as plsc

N_NODES = 10000
D = 128
NC = 2
NS = 16
NW = NC * NS
CHUNK = 128
NBUF = 4
N_ACC = 10240
PAD_DST = N_NODES + 8
SLC = N_ACC // NS

_mesh = plsc.VectorSubcoreMesh(core_axis_name="c", subcore_axis_name="s")


def _make_deg_kernel(nch):
    LAG = 8

    @functools.partial(
        pl.kernel,
        out_type=[jax.ShapeDtypeStruct((N_ACC,), jnp.float32),
                  jax.ShapeDtypeStruct((N_ACC,), jnp.float32)],
        mesh=_mesh,
        scratch_types=[
            pltpu.VMEM((nch, CHUNK), jnp.int32),
            pltpu.VMEM((CHUNK,), jnp.float32),
            pltpu.VMEM((SLC,), jnp.float32),
            pltpu.VMEM_SHARED((N_ACC,), jnp.float32),
            pltpu.SemaphoreType.DMA,
        ],
    )
    def deg_kernel(dst3_hbm, out0_hbm, out1_hbm, didx2, ones_v, zbuf, sdeg,
                   sem):
        c = lax.axis_index("c")
        s = lax.axis_index("s")
        wid = c * NS + s

        def _zero(i, _):
            zbuf[pl.ds(i * 16, 16)] = jnp.zeros((16,), jnp.float32)
            return 0
        lax.fori_loop(0, SLC // 16, _zero, 0)
        for j in range(CHUNK // 16):
            ones_v[pl.ds(j * 16, 16)] = jnp.ones((16,), jnp.float32)
        pltpu.sync_copy(dst3_hbm.at[pl.ds(wid * nch, nch)], didx2)
        pltpu.sync_copy(zbuf, sdeg.at[pl.ds(s * SLC, SLC)])
        plsc.subcore_barrier()

        def _fire(i, _):
            pltpu.async_copy(ones_v, sdeg.at[didx2.at[i]], sem, add=True)

            @pl.when(i >= LAG)
            def _():
                pltpu.make_async_copy(ones_v, sdeg.at[didx2.at[0]],
                                      sem).wait()
            return 0
        lax.fori_loop(0, nch, _fire, 0)

        def _drain(i, _):
            pltpu.make_async_copy(ones_v, sdeg.at[didx2.at[0]], sem).wait()
            return 0
        lax.fori_loop(0, min(LAG, nch), _drain, 0)
        plsc.subcore_barrier()

        pltpu.sync_copy(sdeg.at[pl.ds(s * SLC, SLC)], zbuf)

        @pl.when(c == 0)
        def _():
            pltpu.sync_copy(zbuf, out0_hbm.at[pl.ds(s * SLC, SLC)])

        @pl.when(c == 1)
        def _():
            pltpu.sync_copy(zbuf, out1_hbm.at[pl.ds(s * SLC, SLC)])

    return deg_kernel


ECH = 64
NB = 4


def _make_scatter_kernel(nch):
    assert nch % NB == 0

    @functools.partial(
        pl.kernel,
        out_type=[jax.ShapeDtypeStruct((N_ACC, D), jnp.float32),
                  jax.ShapeDtypeStruct((N_ACC, D), jnp.float32)],
        mesh=_mesh,
        scratch_types=[
            pltpu.VMEM((nch // 2, 2 * ECH), jnp.int32),
            pltpu.VMEM((ECH,), jnp.int32),
            pltpu.VMEM((ECH,), jnp.int32),
            pltpu.VMEM((ECH,), jnp.int32),
            pltpu.VMEM((ECH,), jnp.int32),
            pltpu.VMEM((ECH, D), jnp.float32),
            pltpu.VMEM((ECH, D), jnp.float32),
            pltpu.VMEM((ECH, D), jnp.float32),
            pltpu.VMEM((ECH, D), jnp.float32),
            pltpu.VMEM_SHARED((N_ACC, D), jnp.float32),
            pltpu.SemaphoreType.DMA,
            pltpu.SemaphoreType.DMA,
            pltpu.SemaphoreType.DMA,
            pltpu.SemaphoreType.DMA,
            pltpu.SemaphoreType.DMA,
            pltpu.SemaphoreType.DMA,
            pltpu.SemaphoreType.DMA,
            pltpu.SemaphoreType.DMA,
            pltpu.SemaphoreType.DMA,
            pltpu.SemaphoreType.DMA,
            pltpu.SemaphoreType.DMA,
            pltpu.SemaphoreType.DMA,
        ],
    )
    def scatter_kernel(g_hbm, src3_hbm, dst3_hbm, out0_hbm, out1_hbm,
                       didx2, sx0, sx1, sx2, sx3, r0, r1, r2, r3, acc,
                       is0, is1, is2, is3, gs0, gs1, gs2, gs3,
                       ss0, ss1, ss2, ss3):
        sidx = [sx0, sx1, sx2, sx3]
        rows = [r0, r1, r2, r3]
        isem = [is0, is1, is2, is3]
        gsem = [gs0, gs1, gs2, gs3]
        ssem = [ss0, ss1, ss2, ss3]
        c = lax.axis_index("c")
        s = lax.axis_index("s")
        wid = c * NS + s
        base = wid * nch

        def _zero(i, _):
            r0[i // 8, pl.ds((i % 8) * 16, 16)] = jnp.zeros((16,),
                                                            jnp.float32)
            return 0
        lax.fori_loop(0, ECH * (D // 16), _zero, 0)

        pltpu.sync_copy(dst3_hbm.at[pl.ds(wid * (nch // 2), nch // 2)],
                        didx2)
        for j in range(SLC // ECH):
            pltpu.sync_copy(r0, acc.at[pl.ds(s * SLC + j * ECH, ECH)])
        plsc.subcore_barrier()

        for k in range(NB):
            pltpu.async_copy(src3_hbm.at[base + k], sidx[k], isem[k])

        def _wait_i(k):
            pltpu.make_async_copy(src3_hbm.at[base], sidx[k], isem[k]).wait()

        def _wait_g(k):
            pltpu.make_async_copy(g_hbm.at[sidx[0]], rows[k], gsem[k]).wait()

        def _didx(i, half):
            return didx2.at[i // 2, pl.ds(half * ECH, ECH)]

        def _wait_s(k):
            pltpu.make_async_copy(rows[k], acc.at[_didx(0, 0)],
                                  ssem[k]).wait()

        def _outer(io, _):
            for k in range(NB):
                i = io * NB + k

                @pl.when(i >= NB)
                def _():
                    _wait_s(k)

                @pl.when((i >= 2) & (i + 2 < nch))
                def _():
                    pltpu.async_copy(src3_hbm.at[base + i + 2],
                                     sidx[(k + 2) % NB], isem[(k + 2) % NB])
                _wait_i(k)
                pltpu.async_copy(g_hbm.at[sidx[k]], rows[k], gsem[k])

                @pl.when(i >= 1)
                def _():
                    kp = (k - 1) % NB
                    _wait_g(kp)
                    pltpu.async_copy(rows[kp],
                                     acc.at[_didx(i - 1, (k - 1) % 2)],
                                     ssem[kp], add=True)
            return 0
        lax.fori_loop(0, nch // NB, _outer, 0)

        kl = (nch - 1) % NB
        _wait_g(kl)
        pltpu.async_copy(rows[kl], acc.at[_didx(nch - 1, (nch - 1) % 2)],
                         ssem[kl], add=True)
        for k in range(NB):
            _wait_s(k)
        plsc.subcore_barrier()

        for j in range(SLC // ECH):
            off = s * SLC + j * ECH
            pltpu.sync_copy(acc.at[pl.ds(off, ECH)], r0)

            @pl.when(c == 0)
            def _():
                pltpu.sync_copy(r0, out0_hbm.at[pl.ds(off, ECH)])

            @pl.when(c == 1)
            def _():
                pltpu.sync_copy(r0, out1_hbm.at[pl.ds(off, ECH)])

    return scatter_kernel


BLK = 400


def _matmul_body(x_ref, w_ref, degp_ref, g_ref):
    deg = degp_ref[:, 0] + degp_ref[:, 1] + 1.0
    dis = lax.rsqrt(deg)
    h = jnp.dot(x_ref[...], w_ref[...], preferred_element_type=jnp.float32)
    g_ref[...] = h * dis[:, None]


def _matmul(x, w, degp_t):
    return pl.pallas_call(
        _matmul_body,
        grid=(N_NODES // BLK,),
        in_specs=[
            pl.BlockSpec((BLK, D), lambda i: (i, 0)),
            pl.BlockSpec((D, D), lambda i: (0, 0)),
            pl.BlockSpec((BLK, NC), lambda i: (i, 0)),
        ],
        out_specs=pl.BlockSpec((BLK, D), lambda i: (i, 0)),
        out_shape=jax.ShapeDtypeStruct((N_NODES, D), jnp.float32),
    )(x, w, degp_t)


def _final_body(p0_ref, p1_ref, g_ref, degp_ref, b_ref, o_ref):
    deg = degp_ref[:, 0] + degp_ref[:, 1] + 1.0
    dis = lax.rsqrt(deg)
    o_ref[...] = (dis[:, None] * (p0_ref[...] + p1_ref[...] + g_ref[...])
                  + b_ref[...])


def _final(p0, p1, g, degp_t, b2d):
    return pl.pallas_call(
        _final_body,
        grid=(N_NODES // BLK,),
        in_specs=[
            pl.BlockSpec((BLK, D), lambda i: (i, 0)),
            pl.BlockSpec((BLK, D), lambda i: (i, 0)),
            pl.BlockSpec((BLK, D), lambda i: (i, 0)),
            pl.BlockSpec((BLK, NC), lambda i: (i, 0)),
            pl.BlockSpec((1, D), lambda i: (0, 0)),
        ],
        out_specs=pl.BlockSpec((BLK, D), lambda i: (i, 0)),
        out_shape=jax.ShapeDtypeStruct((N_NODES, D), jnp.float32),
    )(p0, p1, g, degp_t, b2d)


def kernel(node_features, adjacency_matrix, W, b):
    src = adjacency_matrix[0].astype(jnp.int32)
    dst = adjacency_matrix[1].astype(jnp.int32)
    n_edges = src.shape[0]
    quantum = NW * CHUNK * 8
    n_pad = (-n_edges) % quantum
    if n_pad:
        src = jnp.concatenate([src, jnp.zeros((n_pad,), jnp.int32)])
        dst = jnp.concatenate([dst, jnp.full((n_pad,), PAD_DST, jnp.int32)])
    n_tot = n_edges + n_pad
    nch = n_tot // (NW * CHUNK)
    nch_sc = n_tot // (NW * ECH)
    dst3 = dst.reshape(NW * nch, CHUNK)
    src4 = src.reshape(NW * nch_sc, ECH)
    dst4 = dst.reshape(NW * nch_sc // 2, 2 * ECH)

    d0, d1 = _make_deg_kernel(nch)(dst3)
    degp_t = jnp.stack([d0[:N_NODES], d1[:N_NODES]], axis=1)
    g = _matmul(node_features, W, degp_t)
    p0, p1 = _make_scatter_kernel(nch_sc)(g, src4, dst4)
    return _final(p0, p1, g, degp_t, b.reshape(1, D))
